# Initial kernel scaffold; baseline (speedup 1.0000x reference)
import jax
import jax.numpy as jnp
from jax.experimental import pallas as pl


def kernel(x, edge_index, edge_attr, We0, be0, We1, be1, We2, be2, eg, ebe, Wn0, bn0, Wn1, bn1, Wn2, bn2, ng, nbe):
    raise NotImplementedError("write your pallas kernel here")


# speedup vs baseline: 1.1255x; 1.1255x over previous
import functools

import jax
import jax.numpy as jnp
from jax import lax
from jax.experimental import pallas as pl
from jax.experimental.pallas import tpu as pltpu

N = 10000
E = 160000
D = 128
H = 128
NB = 3

TN = 2000
TE = 2000
_F32 = jnp.float32


def _ln_res(base, h, g, b):
    mu = jnp.mean(h, axis=-1, keepdims=True)
    v = jnp.mean((h - mu) ** 2, axis=-1, keepdims=True)
    return base + (h - mu) / jnp.sqrt(v + 1e-5) * g + b


def _pq_body(x_ref, ws_ref, wd_ref, p_ref, q_ref):
    x = x_ref[...]
    p_ref[...] = jnp.dot(x, ws_ref[...], preferred_element_type=_F32)
    q_ref[...] = jnp.dot(x, wd_ref[...], preferred_element_type=_F32)


def _pq_call(x, ws, wd):
    grid = (N // TN,)
    row = pl.BlockSpec((TN, D), lambda i: (i, 0))
    full = pl.BlockSpec((D, H), lambda i: (0, 0))
    return pl.pallas_call(
        _pq_body,
        grid=grid,
        in_specs=[row, full, full],
        out_specs=[row, row],
        out_shape=[jax.ShapeDtypeStruct((N, H), _F32),
                   jax.ShapeDtypeStruct((N, H), _F32)],
    )(x, ws, wd)


def _edge_body(pre_ref, e_ref, w0_ref, b0_ref, w1_ref, b1_ref, w2_ref,
               b2_ref, g_ref, be_ref, out_ref):
    e = e_ref[...]
    h = jnp.dot(e, w0_ref[...], preferred_element_type=_F32)
    h = jnp.maximum(h + pre_ref[...] + b0_ref[...], 0.0)
    h = jnp.maximum(jnp.dot(h, w1_ref[...], preferred_element_type=_F32)
                    + b1_ref[...], 0.0)
    h = jnp.dot(h, w2_ref[...], preferred_element_type=_F32) + b2_ref[...]
    out_ref[...] = _ln_res(e, h, g_ref[...], be_ref[...])


def _edge_call(pre, e, w0, b0, w1, b1, w2, b2, g, be):
    grid = (E // TE,)
    row = pl.BlockSpec((TE, D), lambda i: (i, 0))
    wfull = pl.BlockSpec((H, H), lambda i: (0, 0))
    vfull = pl.BlockSpec((1, H), lambda i: (0, 0))
    return pl.pallas_call(
        _edge_body,
        grid=grid,
        in_specs=[row, row, wfull, vfull, wfull, vfull, wfull, vfull,
                  vfull, vfull],
        out_specs=row,
        out_shape=jax.ShapeDtypeStruct((E, D), _F32),
    )(pre, e, w0, b0, w1, b1, w2, b2, g, be)


def _node_body(has_next, x_ref, agg_ref, w0x_ref, w0a_ref, b0_ref, w1_ref,
               b1_ref, w2_ref, b2_ref, g_ref, be_ref, *rest):
    if has_next:
        ws_ref, wd_ref, xn_ref, p_ref, q_ref = rest
    else:
        (xn_ref,) = rest
    x = x_ref[...]
    h = (jnp.dot(x, w0x_ref[...], preferred_element_type=_F32)
         + jnp.dot(agg_ref[...], w0a_ref[...], preferred_element_type=_F32)
         + b0_ref[...])
    h = jnp.maximum(h, 0.0)
    h = jnp.maximum(jnp.dot(h, w1_ref[...], preferred_element_type=_F32)
                    + b1_ref[...], 0.0)
    h = jnp.dot(h, w2_ref[...], preferred_element_type=_F32) + b2_ref[...]
    xn = _ln_res(x, h, g_ref[...], be_ref[...])
    xn_ref[...] = xn
    if has_next:
        p_ref[...] = jnp.dot(xn, ws_ref[...], preferred_element_type=_F32)
        q_ref[...] = jnp.dot(xn, wd_ref[...], preferred_element_type=_F32)


def _node_call(x, agg, w0x, w0a, b0, w1, b1, w2, b2, g, be, ws_next, wd_next):
    grid = (N // TN,)
    row = pl.BlockSpec((TN, D), lambda i: (i, 0))
    wfull = pl.BlockSpec((H, H), lambda i: (0, 0))
    vfull = pl.BlockSpec((1, H), lambda i: (0, 0))
    has_next = ws_next is not None
    ins = [x, agg, w0x, w0a, b0, w1, b1, w2, b2, g, be]
    in_specs = [row, row, wfull, wfull, vfull, wfull, vfull, wfull, vfull,
                vfull, vfull]
    if has_next:
        ins += [ws_next, wd_next]
        in_specs += [wfull, wfull]
        out_specs = [row, row, row]
        out_shape = [jax.ShapeDtypeStruct((N, D), _F32),
                     jax.ShapeDtypeStruct((N, H), _F32),
                     jax.ShapeDtypeStruct((N, H), _F32)]
    else:
        out_specs = row
        out_shape = jax.ShapeDtypeStruct((N, D), _F32)
    return pl.pallas_call(
        functools.partial(_node_body, has_next),
        grid=grid,
        in_specs=in_specs,
        out_specs=out_specs,
        out_shape=out_shape,
    )(*ins)


def kernel(x, edge_index, edge_attr, We0, be0, We1, be1, We2, be2, eg, ebe,
           Wn0, bn0, Wn1, bn1, Wn2, bn2, ng, nbe):
    src = edge_index[0]
    dst = edge_index[1]
    e = edge_attr

    p, q = _pq_call(x, We0[0, :D], We0[0, D:2 * D])
    for i in range(NB):
        pre = jnp.take(p, src, axis=0) + jnp.take(q, dst, axis=0)
        e = _edge_call(pre, e, We0[i, 2 * D:], be0[i][None], We1[i],
                       be1[i][None], We2[i], be2[i][None], eg[i][None],
                       ebe[i][None])
        agg = jax.ops.segment_sum(e, dst, num_segments=N)
        ws_next = We0[i + 1, :D] if i + 1 < NB else None
        wd_next = We0[i + 1, D:2 * D] if i + 1 < NB else None
        out = _node_call(x, agg, Wn0[i, :D], Wn0[i, D:], bn0[i][None],
                         Wn1[i], bn1[i][None], Wn2[i], bn2[i][None],
                         ng[i][None], nbe[i][None], ws_next, wd_next)
        if i + 1 < NB:
            x, p, q = out
        else:
            x = out
    return x


# speedup vs baseline: 1.9632x; 1.7442x over previous
import functools

import jax
import jax.numpy as jnp
from jax import lax
from jax.experimental import pallas as pl
from jax.experimental.pallas import tpu as pltpu
from jax.experimental.pallas import tpu_sc as plsc

N = 10000
E = 160000
D = 128
H = 128
NB = 3

TN = 2000
TE = 2000
_F32 = jnp.float32

NC_SC = 2
NS_SC = 16
NW_SC = NC_SC * NS_SC
CH = 128
NCHUNK = E // CH
KMAX = -(-NCHUNK // NW_SC)

_sc_mesh = plsc.VectorSubcoreMesh(core_axis_name="c", subcore_axis_name="s",
                                  num_cores=NC_SC, num_subcores=NS_SC)


@functools.partial(
    pl.kernel,
    out_type=jax.ShapeDtypeStruct((E, D), _F32),
    mesh=_sc_mesh,
    scratch_types=[
        pltpu.VMEM((CH,), jnp.int32),
        pltpu.VMEM((CH,), jnp.int32),
        pltpu.VMEM((CH, D), _F32),
        pltpu.VMEM((CH, D), _F32),
        pltpu.SemaphoreType.DMA,
    ],
)
def _sc_gather_add(p_hbm, q_hbm, src_hbm, dst_hbm, out_hbm, si, di, rs, rd,
                   sem):
    wid = lax.axis_index("s") * NC_SC + lax.axis_index("c")

    def step(k, carry):
        c = wid + k * NW_SC

        @pl.when(c < NCHUNK)
        def _():
            base = c * CH
            pltpu.sync_copy(src_hbm.at[pl.ds(base, CH)], si)
            pltpu.sync_copy(dst_hbm.at[pl.ds(base, CH)], di)
            cp1 = pltpu.async_copy(p_hbm.at[si], rs, sem)
            cp2 = pltpu.async_copy(q_hbm.at[di], rd, sem)
            cp1.wait()
            cp2.wait()

            def add_row(r, carry2):
                for j in range(D // 16):
                    sl = pl.ds(j * 16, 16)
                    rs[r, sl] = rs[r, sl] + rd[r, sl]
                return carry2

            lax.fori_loop(0, CH, add_row, 0)
            pltpu.sync_copy(rs, out_hbm.at[pl.ds(base, CH)])

        return carry

    lax.fori_loop(0, KMAX, step, 0)


def _ln_res(base, h, g, b):
    mu = jnp.mean(h, axis=-1, keepdims=True)
    v = jnp.mean((h - mu) ** 2, axis=-1, keepdims=True)
    return base + (h - mu) / jnp.sqrt(v + 1e-5) * g + b


def _pq_body(x_ref, ws_ref, wd_ref, p_ref, q_ref):
    x = x_ref[...]
    p_ref[...] = jnp.dot(x, ws_ref[...], preferred_element_type=_F32)
    q_ref[...] = jnp.dot(x, wd_ref[...], preferred_element_type=_F32)


def _pq_call(x, ws, wd):
    grid = (N // TN,)
    row = pl.BlockSpec((TN, D), lambda i: (i, 0))
    full = pl.BlockSpec((D, H), lambda i: (0, 0))
    return pl.pallas_call(
        _pq_body,
        grid=grid,
        in_specs=[row, full, full],
        out_specs=[row, row],
        out_shape=[jax.ShapeDtypeStruct((N, H), _F32),
                   jax.ShapeDtypeStruct((N, H), _F32)],
    )(x, ws, wd)


def _edge_body(pre_ref, e_ref, w0_ref, b0_ref, w1_ref, b1_ref, w2_ref,
               b2_ref, g_ref, be_ref, out_ref):
    e = e_ref[...]
    h = jnp.dot(e, w0_ref[...], preferred_element_type=_F32)
    h = jnp.maximum(h + pre_ref[...] + b0_ref[...], 0.0)
    h = jnp.maximum(jnp.dot(h, w1_ref[...], preferred_element_type=_F32)
                    + b1_ref[...], 0.0)
    h = jnp.dot(h, w2_ref[...], preferred_element_type=_F32) + b2_ref[...]
    out_ref[...] = _ln_res(e, h, g_ref[...], be_ref[...])


def _edge_call(pre, e, w0, b0, w1, b1, w2, b2, g, be):
    grid = (E // TE,)
    row = pl.BlockSpec((TE, D), lambda i: (i, 0))
    wfull = pl.BlockSpec((H, H), lambda i: (0, 0))
    vfull = pl.BlockSpec((1, H), lambda i: (0, 0))
    return pl.pallas_call(
        _edge_body,
        grid=grid,
        in_specs=[row, row, wfull, vfull, wfull, vfull, wfull, vfull,
                  vfull, vfull],
        out_specs=row,
        out_shape=jax.ShapeDtypeStruct((E, D), _F32),
    )(pre, e, w0, b0, w1, b1, w2, b2, g, be)


def _node_body(has_next, x_ref, agg_ref, w0x_ref, w0a_ref, b0_ref, w1_ref,
               b1_ref, w2_ref, b2_ref, g_ref, be_ref, *rest):
    if has_next:
        ws_ref, wd_ref, xn_ref, p_ref, q_ref = rest
    else:
        (xn_ref,) = rest
    x = x_ref[...]
    h = (jnp.dot(x, w0x_ref[...], preferred_element_type=_F32)
         + jnp.dot(agg_ref[...], w0a_ref[...], preferred_element_type=_F32)
         + b0_ref[...])
    h = jnp.maximum(h, 0.0)
    h = jnp.maximum(jnp.dot(h, w1_ref[...], preferred_element_type=_F32)
                    + b1_ref[...], 0.0)
    h = jnp.dot(h, w2_ref[...], preferred_element_type=_F32) + b2_ref[...]
    xn = _ln_res(x, h, g_ref[...], be_ref[...])
    xn_ref[...] = xn
    if has_next:
        p_ref[...] = jnp.dot(xn, ws_ref[...], preferred_element_type=_F32)
        q_ref[...] = jnp.dot(xn, wd_ref[...], preferred_element_type=_F32)


def _node_call(x, agg, w0x, w0a, b0, w1, b1, w2, b2, g, be, ws_next, wd_next):
    grid = (N // TN,)
    row = pl.BlockSpec((TN, D), lambda i: (i, 0))
    wfull = pl.BlockSpec((H, H), lambda i: (0, 0))
    vfull = pl.BlockSpec((1, H), lambda i: (0, 0))
    has_next = ws_next is not None
    ins = [x, agg, w0x, w0a, b0, w1, b1, w2, b2, g, be]
    in_specs = [row, row, wfull, wfull, vfull, wfull, vfull, wfull, vfull,
                vfull, vfull]
    if has_next:
        ins += [ws_next, wd_next]
        in_specs += [wfull, wfull]
        out_specs = [row, row, row]
        out_shape = [jax.ShapeDtypeStruct((N, D), _F32),
                     jax.ShapeDtypeStruct((N, H), _F32),
                     jax.ShapeDtypeStruct((N, H), _F32)]
    else:
        out_specs = row
        out_shape = jax.ShapeDtypeStruct((N, D), _F32)
    return pl.pallas_call(
        functools.partial(_node_body, has_next),
        grid=grid,
        in_specs=in_specs,
        out_specs=out_specs,
        out_shape=out_shape,
    )(*ins)


def kernel(x, edge_index, edge_attr, We0, be0, We1, be1, We2, be2, eg, ebe,
           Wn0, bn0, Wn1, bn1, Wn2, bn2, ng, nbe):
    src = edge_index[0]
    dst = edge_index[1]
    e = edge_attr

    p, q = _pq_call(x, We0[0, :D], We0[0, D:2 * D])
    for i in range(NB):
        pre = _sc_gather_add(p, q, src, dst)
        e = _edge_call(pre, e, We0[i, 2 * D:], be0[i][None], We1[i],
                       be1[i][None], We2[i], be2[i][None], eg[i][None],
                       ebe[i][None])
        agg = jax.ops.segment_sum(e, dst, num_segments=N)
        ws_next = We0[i + 1, :D] if i + 1 < NB else None
        wd_next = We0[i + 1, D:2 * D] if i + 1 < NB else None
        out = _node_call(x, agg, Wn0[i, :D], Wn0[i, D:], bn0[i][None],
                         Wn1[i], bn1[i][None], Wn2[i], bn2[i][None],
                         ng[i][None], nbe[i][None], ws_next, wd_next)
        if i + 1 < NB:
            x, p, q = out
        else:
            x = out
    return x


# speedup vs baseline: 3.0389x; 1.5479x over previous
import functools

import jax
import jax.numpy as jnp
from jax import lax
from jax.experimental import pallas as pl
from jax.experimental.pallas import tpu as pltpu
from jax.experimental.pallas import tpu_sc as plsc

N = 10000
E = 160000
D = 128
H = 128
NB = 3

TN = 2000
TE = 2000
_F32 = jnp.float32

NC_SC = 2
NS_SC = 16
NW_SC = NC_SC * NS_SC
CH = 128
NCHUNK = E // CH
KMAX = -(-NCHUNK // NW_SC)

_sc_mesh = plsc.VectorSubcoreMesh(core_axis_name="c", subcore_axis_name="s",
                                  num_cores=NC_SC, num_subcores=NS_SC)


@functools.partial(
    pl.kernel,
    out_type=jax.ShapeDtypeStruct((E, D), _F32),
    mesh=_sc_mesh,
    scratch_types=[
        pltpu.VMEM((CH,), jnp.int32),
        pltpu.VMEM((CH,), jnp.int32),
        pltpu.VMEM((CH, D), _F32),
        pltpu.VMEM((CH, D), _F32),
        pltpu.SemaphoreType.DMA,
    ],
)
def _sc_gather_add(p_hbm, q_hbm, src_hbm, dst_hbm, out_hbm, si, di, rs, rd,
                   sem):
    wid = lax.axis_index("s") * NC_SC + lax.axis_index("c")

    def step(k, carry):
        c = wid + k * NW_SC

        @pl.when(c < NCHUNK)
        def _():
            base = c * CH
            pltpu.sync_copy(src_hbm.at[pl.ds(base, CH)], si)
            pltpu.sync_copy(dst_hbm.at[pl.ds(base, CH)], di)
            cp1 = pltpu.async_copy(p_hbm.at[si], rs, sem)
            cp2 = pltpu.async_copy(q_hbm.at[di], rd, sem)
            cp1.wait()
            cp2.wait()

            def add_row(r, carry2):
                for j in range(D // 16):
                    sl = pl.ds(j * 16, 16)
                    rs[r, sl] = rs[r, sl] + rd[r, sl]
                return carry2

            lax.fori_loop(0, CH, add_row, 0)
            pltpu.sync_copy(rs, out_hbm.at[pl.ds(base, CH)])

        return carry

    lax.fori_loop(0, KMAX, step, 0)


def _ln_res(base, h, g, b):
    mu = jnp.mean(h, axis=-1, keepdims=True)
    v = jnp.mean((h - mu) ** 2, axis=-1, keepdims=True)
    return base + (h - mu) / jnp.sqrt(v + 1e-5) * g + b


NCHUNK_C = NCHUNK // NC_SC
KMAX_C = -(-NCHUNK_C // NS_SC)
N_PAD = 10240
NROW_S = N_PAD // NS_SC
RB = 128


@functools.partial(
    pl.kernel,
    out_type=jax.ShapeDtypeStruct((NC_SC, N_PAD, D), _F32),
    mesh=_sc_mesh,
    scratch_types=[
        pltpu.VMEM((CH,), jnp.int32),
        pltpu.VMEM((CH, D), _F32),
        pltpu.VMEM((RB, D), _F32),
        pltpu.VMEM_SHARED((N_PAD, D), _F32),
    ],
)
def _sc_scatter_add(e_hbm, dst_hbm, out_hbm, di, rows, stage, acc_sh):
    cid = lax.axis_index("c")
    sid = lax.axis_index("s")

    def zrow(r, carry):
        for j in range(D // 16):
            stage[r, pl.ds(j * 16, 16)] = jnp.zeros((16,), _F32)
        return carry

    lax.fori_loop(0, RB, zrow, 0)
    for t in range(NROW_S // RB):
        pltpu.sync_copy(stage, acc_sh.at[pl.ds(sid * NROW_S + t * RB, RB)])
    plsc.subcore_barrier()

    def step(k, carry):
        c = sid + k * NS_SC

        @pl.when(c < NCHUNK_C)
        def _():
            base = (cid * NCHUNK_C + c) * CH
            pltpu.sync_copy(dst_hbm.at[pl.ds(base, CH)], di)
            pltpu.sync_copy(e_hbm.at[pl.ds(base, CH)], rows)
            pltpu.sync_copy(rows, acc_sh.at[di], add=True)

        return carry

    lax.fori_loop(0, KMAX_C, step, 0)
    plsc.subcore_barrier()

    for t in range(NROW_S // RB):
        row = sid * NROW_S + t * RB
        pltpu.sync_copy(acc_sh.at[pl.ds(row, RB)], stage)
        pltpu.sync_copy(stage, out_hbm.at[cid, pl.ds(row, RB)])


def _pq_body(x_ref, ws_ref, wd_ref, p_ref, q_ref):
    x = x_ref[...]
    p_ref[...] = jnp.dot(x, ws_ref[...], preferred_element_type=_F32)
    q_ref[...] = jnp.dot(x, wd_ref[...], preferred_element_type=_F32)


def _pq_call(x, ws, wd):
    grid = (N // TN,)
    row = pl.BlockSpec((TN, D), lambda i: (i, 0))
    full = pl.BlockSpec((D, H), lambda i: (0, 0))
    return pl.pallas_call(
        _pq_body,
        grid=grid,
        in_specs=[row, full, full],
        out_specs=[row, row],
        out_shape=[jax.ShapeDtypeStruct((N, H), _F32),
                   jax.ShapeDtypeStruct((N, H), _F32)],
    )(x, ws, wd)


def _edge_body(pre_ref, e_ref, w0_ref, b0_ref, w1_ref, b1_ref, w2_ref,
               b2_ref, g_ref, be_ref, out_ref):
    e = e_ref[...]
    h = jnp.dot(e, w0_ref[...], preferred_element_type=_F32)
    h = jnp.maximum(h + pre_ref[...] + b0_ref[...], 0.0)
    h = jnp.maximum(jnp.dot(h, w1_ref[...], preferred_element_type=_F32)
                    + b1_ref[...], 0.0)
    h = jnp.dot(h, w2_ref[...], preferred_element_type=_F32) + b2_ref[...]
    out_ref[...] = _ln_res(e, h, g_ref[...], be_ref[...])


def _edge_call(pre, e, w0, b0, w1, b1, w2, b2, g, be):
    grid = (E // TE,)
    row = pl.BlockSpec((TE, D), lambda i: (i, 0))
    wfull = pl.BlockSpec((H, H), lambda i: (0, 0))
    vfull = pl.BlockSpec((1, H), lambda i: (0, 0))
    return pl.pallas_call(
        _edge_body,
        grid=grid,
        in_specs=[row, row, wfull, vfull, wfull, vfull, wfull, vfull,
                  vfull, vfull],
        out_specs=row,
        out_shape=jax.ShapeDtypeStruct((E, D), _F32),
    )(pre, e, w0, b0, w1, b1, w2, b2, g, be)


def _node_body(has_next, x_ref, a0_ref, a1_ref, w0x_ref, w0a_ref, b0_ref,
               w1_ref, b1_ref, w2_ref, b2_ref, g_ref, be_ref, *rest):
    if has_next:
        ws_ref, wd_ref, xn_ref, p_ref, q_ref = rest
    else:
        (xn_ref,) = rest
    x = x_ref[...]
    agg = a0_ref[...] + a1_ref[...]
    h = (jnp.dot(x, w0x_ref[...], preferred_element_type=_F32)
         + jnp.dot(agg, w0a_ref[...], preferred_element_type=_F32)
         + b0_ref[...])
    h = jnp.maximum(h, 0.0)
    h = jnp.maximum(jnp.dot(h, w1_ref[...], preferred_element_type=_F32)
                    + b1_ref[...], 0.0)
    h = jnp.dot(h, w2_ref[...], preferred_element_type=_F32) + b2_ref[...]
    xn = _ln_res(x, h, g_ref[...], be_ref[...])
    xn_ref[...] = xn
    if has_next:
        p_ref[...] = jnp.dot(xn, ws_ref[...], preferred_element_type=_F32)
        q_ref[...] = jnp.dot(xn, wd_ref[...], preferred_element_type=_F32)


def _node_call(x, a0, a1, w0x, w0a, b0, w1, b1, w2, b2, g, be, ws_next,
               wd_next):
    grid = (N // TN,)
    row = pl.BlockSpec((TN, D), lambda i: (i, 0))
    wfull = pl.BlockSpec((H, H), lambda i: (0, 0))
    vfull = pl.BlockSpec((1, H), lambda i: (0, 0))
    has_next = ws_next is not None
    ins = [x, a0, a1, w0x, w0a, b0, w1, b1, w2, b2, g, be]
    in_specs = [row, row, row, wfull, wfull, vfull, wfull, vfull, wfull,
                vfull, vfull, vfull]
    if has_next:
        ins += [ws_next, wd_next]
        in_specs += [wfull, wfull]
        out_specs = [row, row, row]
        out_shape = [jax.ShapeDtypeStruct((N, D), _F32),
                     jax.ShapeDtypeStruct((N, H), _F32),
                     jax.ShapeDtypeStruct((N, H), _F32)]
    else:
        out_specs = row
        out_shape = jax.ShapeDtypeStruct((N, D), _F32)
    return pl.pallas_call(
        functools.partial(_node_body, has_next),
        grid=grid,
        in_specs=in_specs,
        out_specs=out_specs,
        out_shape=out_shape,
    )(*ins)


def kernel(x, edge_index, edge_attr, We0, be0, We1, be1, We2, be2, eg, ebe,
           Wn0, bn0, Wn1, bn1, Wn2, bn2, ng, nbe):
    src = edge_index[0]
    dst = edge_index[1]
    e = edge_attr

    p, q = _pq_call(x, We0[0, :D], We0[0, D:2 * D])
    for i in range(NB):
        pre = _sc_gather_add(p, q, src, dst)
        e = _edge_call(pre, e, We0[i, 2 * D:], be0[i][None], We1[i],
                       be1[i][None], We2[i], be2[i][None], eg[i][None],
                       ebe[i][None])
        parts = _sc_scatter_add(e, dst)
        ws_next = We0[i + 1, :D] if i + 1 < NB else None
        wd_next = We0[i + 1, D:2 * D] if i + 1 < NB else None
        out = _node_call(x, parts[0], parts[1], Wn0[i, :D], Wn0[i, D:],
                         bn0[i][None],
                         Wn1[i], bn1[i][None], Wn2[i], bn2[i][None],
                         ng[i][None], nbe[i][None], ws_next, wd_next)
        if i + 1 < NB:
            x, p, q = out
        else:
            x = out
    return x


# speedup vs baseline: 4.3968x; 1.4468x over previous
import functools

import jax
import jax.numpy as jnp
from jax import lax
from jax.experimental import pallas as pl
from jax.experimental.pallas import tpu as pltpu
from jax.experimental.pallas import tpu_sc as plsc

N = 10000
E = 160000
D = 128
H = 128
NB = 3

TN = 2000
TE = 2000
_F32 = jnp.float32

NC_SC = 2
NS_SC = 16
NW_SC = NC_SC * NS_SC
CH = 128
NCHUNK = E // CH
KMAX = -(-NCHUNK // NW_SC)

_sc_mesh = plsc.VectorSubcoreMesh(core_axis_name="c", subcore_axis_name="s",
                                  num_cores=NC_SC, num_subcores=NS_SC)


GW = 40
GEW = GW * CH
GPAD = GW * CH * NW_SC
T_LAST = NCHUNK - GW * (NW_SC - 1)


@functools.partial(
    pl.kernel,
    out_type=jax.ShapeDtypeStruct((E, D), _F32),
    mesh=_sc_mesh,
    scratch_types=[
        pltpu.VMEM((GEW,), jnp.int32),
        pltpu.VMEM((GEW,), jnp.int32),
        pltpu.VMEM((2, CH, D), _F32),
        pltpu.VMEM((2, CH, D), _F32),
        pltpu.SemaphoreType.DMA,
        pltpu.SemaphoreType.DMA,
        pltpu.SemaphoreType.DMA,
        pltpu.SemaphoreType.DMA,
        pltpu.SemaphoreType.DMA,
    ],
)
def _sc_gather_add(p_hbm, q_hbm, src_hbm, dst_hbm, out_hbm, si, di, rs, rd,
                   gs0, gs1, ws0, ws1, isem):
    wid = lax.axis_index("s") * NC_SC + lax.axis_index("c")
    t_w = jnp.where(wid == NW_SC - 1, T_LAST, GW)
    ebase = wid * GEW
    c1 = pltpu.async_copy(src_hbm.at[pl.ds(ebase, GEW)], si, isem)
    c2 = pltpu.async_copy(dst_hbm.at[pl.ds(ebase, GEW)], di, isem)
    c1.wait()
    c2.wait()

    gsems = (gs0, gs1)
    wsems = (ws0, ws1)

    def fire(c, b):
        off = pl.multiple_of(c * CH, CH)
        pltpu.async_copy(p_hbm.at[si.at[pl.ds(off, CH)]], rs.at[b], gsems[b])
        pltpu.async_copy(q_hbm.at[di.at[pl.ds(off, CH)]], rd.at[b], gsems[b])

    def wait_g(b):
        pltpu.make_async_copy(p_hbm.at[pl.ds(0, CH)], rs.at[b],
                              gsems[b]).wait()
        pltpu.make_async_copy(q_hbm.at[pl.ds(0, CH)], rd.at[b],
                              gsems[b]).wait()

    def add(b):
        rb = rs.at[b]
        db = rd.at[b]

        def row(r, carry2):
            for j in range(D // 16):
                sl = pl.ds(j * 16, 16)
                rb[r, sl] = rb[r, sl] + db[r, sl]
            return carry2

        lax.fori_loop(0, CH, row, 0)

    def fire_write(c, b):
        off = pl.multiple_of(c * CH, CH)
        pltpu.async_copy(rs.at[b], out_hbm.at[pl.ds(ebase + off, CH)],
                         wsems[b])

    def wait_w(b):
        pltpu.make_async_copy(rs.at[b], out_hbm.at[pl.ds(0, CH)],
                              wsems[b]).wait()

    fire(0, 0)
    fire(1, 1)

    def pair(k2, carry):
        for b in (0, 1):
            c = 2 * k2 + b
            wait_g(b)
            add(b)
            fire_write(c, b)
            nxt = c + 2

            @pl.when(nxt < t_w)
            def _():
                wait_w(b)
                fire(nxt, b)
        return carry

    lax.fori_loop(0, t_w // 2, pair, 0)
    wait_w(0)
    wait_w(1)


def _ln_res(base, h, g, b):
    mu = jnp.mean(h, axis=-1, keepdims=True)
    v = jnp.mean((h - mu) ** 2, axis=-1, keepdims=True)
    return base + (h - mu) / jnp.sqrt(v + 1e-5) * g + b


NCHUNK_C = NCHUNK // NC_SC
KMAX_C = -(-NCHUNK_C // NS_SC)
N_PAD = 10240
NROW_S = N_PAD // NS_SC
RB = 128


@functools.partial(
    pl.kernel,
    out_type=jax.ShapeDtypeStruct((NC_SC, N_PAD, D), _F32),
    mesh=_sc_mesh,
    scratch_types=[
        pltpu.VMEM((GW, CH), jnp.int32),
        pltpu.VMEM((2, CH, D), _F32),
        pltpu.VMEM_SHARED((N_PAD, D), _F32),
        pltpu.SemaphoreType.DMA,
        pltpu.SemaphoreType.DMA,
        pltpu.SemaphoreType.DMA,
        pltpu.SemaphoreType.DMA,
        pltpu.SemaphoreType.DMA,
    ],
)
def _sc_scatter_add(e_hbm, dstr_hbm, out_hbm, di, rows, acc_sh,
                    fs0, fs1, ss0, ss1, osem):
    cid = lax.axis_index("c")
    sid = lax.axis_index("s")
    wid = sid * NC_SC + cid
    t_w = jnp.where(wid == NW_SC - 1, T_LAST, GW)
    cbase = wid * GW

    ic = pltpu.async_copy(dstr_hbm.at[pl.ds(cbase, GW)], di, osem)

    zb = rows.at[0]

    def zrow(r, carry):
        for j in range(D // 16):
            zb[r, pl.ds(j * 16, 16)] = jnp.zeros((16,), _F32)
        return carry

    lax.fori_loop(0, RB, zrow, 0)
    for t in range(NROW_S // RB):
        pltpu.sync_copy(zb, acc_sh.at[pl.ds(sid * NROW_S + t * RB, RB)])
    ic.wait()
    plsc.subcore_barrier()

    fsems = (fs0, fs1)
    ssems = (ss0, ss1)

    def fire_fill(c, b):
        base = pl.multiple_of((cbase + c) * CH, CH)
        pltpu.async_copy(e_hbm.at[pl.ds(base, CH)], rows.at[b], fsems[b])

    def wait_fill(b):
        pltpu.make_async_copy(e_hbm.at[pl.ds(0, CH)], rows.at[b],
                              fsems[b]).wait()

    def wait_scat(b):
        pltpu.make_async_copy(rows.at[b], acc_sh.at[pl.ds(0, CH)],
                              ssems[b]).wait()

    fire_fill(0, 0)

    def pair(k2, carry):
        for b in (0, 1):
            c = 2 * k2 + b
            wait_fill(b)
            pltpu.async_copy(rows.at[b], acc_sh.at[di.at[c]], ssems[b],
                             add=True)
            nb = 1 - b

            @pl.when(c + 1 < t_w)
            def _():
                @pl.when(c >= 1)
                def _():
                    wait_scat(nb)

                fire_fill(c + 1, nb)
        return carry

    lax.fori_loop(0, t_w // 2, pair, 0)
    wait_scat(0)
    wait_scat(1)
    plsc.subcore_barrier()

    nhop = NROW_S // RB
    for t in range(nhop):
        b = t % 2
        if t >= 2:
            pltpu.make_async_copy(rows.at[b], out_hbm.at[0, pl.ds(0, RB)],
                                  osem).wait()
        row = sid * NROW_S + t * RB
        pltpu.sync_copy(acc_sh.at[pl.ds(row, RB)], rows.at[b])
        pltpu.async_copy(rows.at[b], out_hbm.at[cid, pl.ds(row, RB)], osem)
    for t in range(min(2, nhop)):
        pltpu.make_async_copy(rows.at[t], out_hbm.at[0, pl.ds(0, RB)],
                              osem).wait()


def _pq_body(x_ref, ws_ref, wd_ref, p_ref, q_ref):
    x = x_ref[...]
    p_ref[...] = jnp.dot(x, ws_ref[...], preferred_element_type=_F32)
    q_ref[...] = jnp.dot(x, wd_ref[...], preferred_element_type=_F32)


def _pq_call(x, ws, wd):
    grid = (N // TN,)
    row = pl.BlockSpec((TN, D), lambda i: (i, 0))
    full = pl.BlockSpec((D, H), lambda i: (0, 0))
    return pl.pallas_call(
        _pq_body,
        grid=grid,
        in_specs=[row, full, full],
        out_specs=[row, row],
        out_shape=[jax.ShapeDtypeStruct((N, H), _F32),
                   jax.ShapeDtypeStruct((N, H), _F32)],
    )(x, ws, wd)


def _edge_body(pre_ref, e_ref, w0_ref, b0_ref, w1_ref, b1_ref, w2_ref,
               b2_ref, g_ref, be_ref, out_ref):
    e = e_ref[...]
    h = jnp.dot(e, w0_ref[...], preferred_element_type=_F32)
    h = jnp.maximum(h + pre_ref[...] + b0_ref[...], 0.0)
    h = jnp.maximum(jnp.dot(h, w1_ref[...], preferred_element_type=_F32)
                    + b1_ref[...], 0.0)
    h = jnp.dot(h, w2_ref[...], preferred_element_type=_F32) + b2_ref[...]
    out_ref[...] = _ln_res(e, h, g_ref[...], be_ref[...])


def _edge_call(pre, e, w0, b0, w1, b1, w2, b2, g, be):
    grid = (E // TE,)
    row = pl.BlockSpec((TE, D), lambda i: (i, 0))
    wfull = pl.BlockSpec((H, H), lambda i: (0, 0))
    vfull = pl.BlockSpec((1, H), lambda i: (0, 0))
    return pl.pallas_call(
        _edge_body,
        grid=grid,
        in_specs=[row, row, wfull, vfull, wfull, vfull, wfull, vfull,
                  vfull, vfull],
        out_specs=row,
        out_shape=jax.ShapeDtypeStruct((E, D), _F32),
    )(pre, e, w0, b0, w1, b1, w2, b2, g, be)


def _node_body(has_next, x_ref, a0_ref, a1_ref, w0x_ref, w0a_ref, b0_ref,
               w1_ref, b1_ref, w2_ref, b2_ref, g_ref, be_ref, *rest):
    if has_next:
        ws_ref, wd_ref, xn_ref, p_ref, q_ref = rest
    else:
        (xn_ref,) = rest
    x = x_ref[...]
    agg = a0_ref[...] + a1_ref[...]
    h = (jnp.dot(x, w0x_ref[...], preferred_element_type=_F32)
         + jnp.dot(agg, w0a_ref[...], preferred_element_type=_F32)
         + b0_ref[...])
    h = jnp.maximum(h, 0.0)
    h = jnp.maximum(jnp.dot(h, w1_ref[...], preferred_element_type=_F32)
                    + b1_ref[...], 0.0)
    h = jnp.dot(h, w2_ref[...], preferred_element_type=_F32) + b2_ref[...]
    xn = _ln_res(x, h, g_ref[...], be_ref[...])
    xn_ref[...] = xn
    if has_next:
        p_ref[...] = jnp.dot(xn, ws_ref[...], preferred_element_type=_F32)
        q_ref[...] = jnp.dot(xn, wd_ref[...], preferred_element_type=_F32)


def _node_call(x, a0, a1, w0x, w0a, b0, w1, b1, w2, b2, g, be, ws_next,
               wd_next):
    grid = (N // TN,)
    row = pl.BlockSpec((TN, D), lambda i: (i, 0))
    wfull = pl.BlockSpec((H, H), lambda i: (0, 0))
    vfull = pl.BlockSpec((1, H), lambda i: (0, 0))
    has_next = ws_next is not None
    ins = [x, a0, a1, w0x, w0a, b0, w1, b1, w2, b2, g, be]
    in_specs = [row, row, row, wfull, wfull, vfull, wfull, vfull, wfull,
                vfull, vfull, vfull]
    if has_next:
        ins += [ws_next, wd_next]
        in_specs += [wfull, wfull]
        out_specs = [row, row, row]
        out_shape = [jax.ShapeDtypeStruct((N, D), _F32),
                     jax.ShapeDtypeStruct((N, H), _F32),
                     jax.ShapeDtypeStruct((N, H), _F32)]
    else:
        out_specs = row
        out_shape = jax.ShapeDtypeStruct((N, D), _F32)
    return pl.pallas_call(
        functools.partial(_node_body, has_next),
        grid=grid,
        in_specs=in_specs,
        out_specs=out_specs,
        out_shape=out_shape,
    )(*ins)


def kernel(x, edge_index, edge_attr, We0, be0, We1, be1, We2, be2, eg, ebe,
           Wn0, bn0, Wn1, bn1, Wn2, bn2, ng, nbe):
    src = edge_index[0]
    dst = edge_index[1]
    src_p = jnp.pad(src, (0, GPAD - E))
    dst_p = jnp.pad(dst, (0, GPAD - E))
    dst_r = dst_p.reshape(GPAD // CH, CH)
    e = edge_attr

    p, q = _pq_call(x, We0[0, :D], We0[0, D:2 * D])
    for i in range(NB):
        pre = _sc_gather_add(p, q, src_p, dst_p)
        e = _edge_call(pre, e, We0[i, 2 * D:], be0[i][None], We1[i],
                       be1[i][None], We2[i], be2[i][None], eg[i][None],
                       ebe[i][None])
        parts = _sc_scatter_add(e, dst_r)
        ws_next = We0[i + 1, :D] if i + 1 < NB else None
        wd_next = We0[i + 1, D:2 * D] if i + 1 < NB else None
        out = _node_call(x, parts[0], parts[1], Wn0[i, :D], Wn0[i, D:],
                         bn0[i][None],
                         Wn1[i], bn1[i][None], Wn2[i], bn2[i][None],
                         ng[i][None], nbe[i][None], ws_next, wd_next)
        if i + 1 < NB:
            x, p, q = out
        else:
            x = out
    return x


# speedup vs baseline: 4.7394x; 1.0779x over previous
import functools

import jax
import jax.numpy as jnp
from jax import lax
from jax.experimental import pallas as pl
from jax.experimental.pallas import tpu as pltpu
from jax.experimental.pallas import tpu_sc as plsc

N = 10000
E = 160000
D = 128
H = 128
NB = 3

TN = 2000
TE = 2000
_F32 = jnp.float32

NC_SC = 2
NS_SC = 16
NW_SC = NC_SC * NS_SC
CH = 128
N_PAD = 10240
NROW_S = N_PAD // NS_SC
RB = 128

NSEG = 2
ESEG = E // NSEG

_sc_mesh = plsc.VectorSubcoreMesh(core_axis_name="c", subcore_axis_name="s",
                                  num_cores=NC_SC, num_subcores=NS_SC)


def _group_geom(esz):
    nchunk = esz // CH
    gw = -(-nchunk // NW_SC)
    t_last = nchunk - gw * (NW_SC - 1)
    return nchunk, gw, gw * CH, t_last


def _mk_gather(esz):
    nchunk, gw, gew, t_last = _group_geom(esz)

    @functools.partial(
        pl.kernel,
        out_type=jax.ShapeDtypeStruct((esz, D), _F32),
        mesh=_sc_mesh,
        scratch_types=[
            pltpu.VMEM((gew,), jnp.int32),
            pltpu.VMEM((gew,), jnp.int32),
            pltpu.VMEM((2, CH, D), _F32),
            pltpu.VMEM((2, CH, D), _F32),
            pltpu.SemaphoreType.DMA,
            pltpu.SemaphoreType.DMA,
            pltpu.SemaphoreType.DMA,
            pltpu.SemaphoreType.DMA,
            pltpu.SemaphoreType.DMA,
        ],
    )
    def gather(p_hbm, q_hbm, src_hbm, dst_hbm, out_hbm, si, di, rs, rd,
               gs0, gs1, ws0, ws1, isem):
        wid = lax.axis_index("s") * NC_SC + lax.axis_index("c")
        t_w = jnp.where(wid == NW_SC - 1, t_last, gw)
        ebase = wid * gew
        c1 = pltpu.async_copy(src_hbm.at[pl.ds(ebase, gew)], si, isem)
        c2 = pltpu.async_copy(dst_hbm.at[pl.ds(ebase, gew)], di, isem)
        c1.wait()
        c2.wait()

        gsems = (gs0, gs1)
        wsems = (ws0, ws1)

        def fire(c, b):
            off = pl.multiple_of(c * CH, CH)
            pltpu.async_copy(p_hbm.at[si.at[pl.ds(off, CH)]], rs.at[b],
                             gsems[b])
            pltpu.async_copy(q_hbm.at[di.at[pl.ds(off, CH)]], rd.at[b],
                             gsems[b])

        def wait_g(b):
            pltpu.make_async_copy(p_hbm.at[pl.ds(0, CH)], rs.at[b],
                                  gsems[b]).wait()
            pltpu.make_async_copy(q_hbm.at[pl.ds(0, CH)], rd.at[b],
                                  gsems[b]).wait()

        def add(b):
            rb = rs.at[b]
            db = rd.at[b]

            def row(r, carry2):
                for j in range(D // 16):
                    sl = pl.ds(j * 16, 16)
                    rb[r, sl] = rb[r, sl] + db[r, sl]
                return carry2

            lax.fori_loop(0, CH, row, 0)

        def fire_write(c, b):
            off = pl.multiple_of(c * CH, CH)
            pltpu.async_copy(rs.at[b], out_hbm.at[pl.ds(ebase + off, CH)],
                             wsems[b])

        def wait_w(b):
            pltpu.make_async_copy(rs.at[b], out_hbm.at[pl.ds(0, CH)],
                                  wsems[b]).wait()

        fire(0, 0)
        fire(1, 1)

        def pair(k2, carry):
            for b in (0, 1):
                c = 2 * k2 + b
                wait_g(b)
                add(b)
                fire_write(c, b)
                nxt = c + 2

                @pl.when(nxt < t_w)
                def _():
                    wait_w(b)
                    fire(nxt, b)
            return carry

        lax.fori_loop(0, t_w // 2, pair, 0)

        @pl.when(t_w % 2 == 1)
        def _():
            wait_g(0)
            add(0)
            fire_write(t_w - 1, 0)

        wait_w(0)
        wait_w(1)

    return gather


def _mk_scatter(esz):
    nchunk, gw, gew, t_last = _group_geom(esz)

    @functools.partial(
        pl.kernel,
        out_type=jax.ShapeDtypeStruct((NC_SC, N_PAD, D), _F32),
        mesh=_sc_mesh,
        scratch_types=[
            pltpu.VMEM((gw, CH), jnp.int32),
            pltpu.VMEM((2, CH, D), _F32),
            pltpu.VMEM_SHARED((N_PAD, D), _F32),
            pltpu.SemaphoreType.DMA,
            pltpu.SemaphoreType.DMA,
            pltpu.SemaphoreType.DMA,
            pltpu.SemaphoreType.DMA,
            pltpu.SemaphoreType.DMA,
        ],
    )
    def scatter(e_hbm, dstr_hbm, out_hbm, di, rows, acc_sh,
                fs0, fs1, ss0, ss1, osem):
        cid = lax.axis_index("c")
        sid = lax.axis_index("s")
        wid = sid * NC_SC + cid
        t_w = jnp.where(wid == NW_SC - 1, t_last, gw)
        cbase = wid * gw

        ic = pltpu.async_copy(dstr_hbm.at[wid], di, osem)

        zb = rows.at[0]

        def zrow(r, carry):
            for j in range(D // 16):
                zb[r, pl.ds(j * 16, 16)] = jnp.zeros((16,), _F32)
            return carry

        lax.fori_loop(0, RB, zrow, 0)
        for t in range(NROW_S // RB):
            pltpu.sync_copy(zb, acc_sh.at[pl.ds(sid * NROW_S + t * RB, RB)])
        ic.wait()
        plsc.subcore_barrier()

        fsems = (fs0, fs1)
        ssems = (ss0, ss1)

        def fire_fill(c, b):
            base = pl.multiple_of((cbase + c) * CH, CH)
            pltpu.async_copy(e_hbm.at[pl.ds(base, CH)], rows.at[b], fsems[b])

        def wait_fill(b):
            pltpu.make_async_copy(e_hbm.at[pl.ds(0, CH)], rows.at[b],
                                  fsems[b]).wait()

        def wait_scat(b):
            pltpu.make_async_copy(rows.at[b], acc_sh.at[pl.ds(0, CH)],
                                  ssems[b]).wait()

        fire_fill(0, 0)

        def pair(k2, carry):
            for b in (0, 1):
                c = 2 * k2 + b
                wait_fill(b)
                pltpu.async_copy(rows.at[b], acc_sh.at[di.at[c]], ssems[b],
                                 add=True)
                nb = 1 - b

                @pl.when(c + 1 < t_w)
                def _():
                    @pl.when(c >= 1)
                    def _():
                        wait_scat(nb)

                    fire_fill(c + 1, nb)
            return carry

        lax.fori_loop(0, t_w // 2, pair, 0)

        @pl.when(t_w % 2 == 1)
        def _():
            wait_fill(0)
            pltpu.async_copy(rows.at[0], acc_sh.at[di.at[t_w - 1]], ss0,
                             add=True)

        wait_scat(0)
        wait_scat(1)
        plsc.subcore_barrier()

        nhop = NROW_S // RB
        for t in range(nhop):
            b = t % 2
            if t >= 2:
                pltpu.make_async_copy(rows.at[b],
                                      out_hbm.at[0, pl.ds(0, RB)],
                                      osem).wait()
            row = sid * NROW_S + t * RB
            pltpu.sync_copy(acc_sh.at[pl.ds(row, RB)], rows.at[b])
            pltpu.async_copy(rows.at[b], out_hbm.at[cid, pl.ds(row, RB)],
                             osem)
        for t in range(min(2, nhop)):
            pltpu.make_async_copy(rows.at[t], out_hbm.at[0, pl.ds(0, RB)],
                                  osem).wait()

    return scatter


_sc_gather_seg = _mk_gather(ESEG)
_sc_scatter_seg = _mk_scatter(ESEG)


def _ln_res(base, h, g, b):
    mu = jnp.mean(h, axis=-1, keepdims=True)
    v = jnp.mean((h - mu) ** 2, axis=-1, keepdims=True)
    return base + (h - mu) / jnp.sqrt(v + 1e-5) * g + b


def _pq_body(x_ref, ws_ref, wd_ref, p_ref, q_ref):
    x = x_ref[...]
    p_ref[...] = jnp.dot(x, ws_ref[...], preferred_element_type=_F32)
    q_ref[...] = jnp.dot(x, wd_ref[...], preferred_element_type=_F32)


def _pq_call(x, ws, wd):
    grid = (N // TN,)
    row = pl.BlockSpec((TN, D), lambda i: (i, 0))
    full = pl.BlockSpec((D, H), lambda i: (0, 0))
    return pl.pallas_call(
        _pq_body,
        grid=grid,
        in_specs=[row, full, full],
        out_specs=[row, row],
        out_shape=[jax.ShapeDtypeStruct((N, H), _F32),
                   jax.ShapeDtypeStruct((N, H), _F32)],
    )(x, ws, wd)


def _edge_body(pre_ref, e_ref, w0_ref, b0_ref, w1_ref, b1_ref, w2_ref,
               b2_ref, g_ref, be_ref, out_ref):
    e = e_ref[...]
    h = jnp.dot(e, w0_ref[...], preferred_element_type=_F32)
    h = jnp.maximum(h + pre_ref[...] + b0_ref[...], 0.0)
    h = jnp.maximum(jnp.dot(h, w1_ref[...], preferred_element_type=_F32)
                    + b1_ref[...], 0.0)
    h = jnp.dot(h, w2_ref[...], preferred_element_type=_F32) + b2_ref[...]
    out_ref[...] = _ln_res(e, h, g_ref[...], be_ref[...])


def _edge_call(pre, e, w0, b0, w1, b1, w2, b2, g, be):
    esz = e.shape[0]
    grid = (esz // TE,)
    row = pl.BlockSpec((TE, D), lambda i: (i, 0))
    wfull = pl.BlockSpec((H, H), lambda i: (0, 0))
    vfull = pl.BlockSpec((1, H), lambda i: (0, 0))
    return pl.pallas_call(
        _edge_body,
        grid=grid,
        in_specs=[row, row, wfull, vfull, wfull, vfull, wfull, vfull,
                  vfull, vfull],
        out_specs=row,
        out_shape=jax.ShapeDtypeStruct((esz, D), _F32),
    )(pre, e, w0, b0, w1, b1, w2, b2, g, be)


def _node_body(has_next, x_ref, a0_ref, a1_ref, a2_ref, a3_ref, w0x_ref,
               w0a_ref, b0_ref, w1_ref, b1_ref, w2_ref, b2_ref, g_ref,
               be_ref, *rest):
    if has_next:
        ws_ref, wd_ref, xn_ref, p_ref, q_ref = rest
    else:
        (xn_ref,) = rest
    x = x_ref[...]
    agg = ((a0_ref[...] + a1_ref[...]) + (a2_ref[...] + a3_ref[...]))
    h = (jnp.dot(x, w0x_ref[...], preferred_element_type=_F32)
         + jnp.dot(agg, w0a_ref[...], preferred_element_type=_F32)
         + b0_ref[...])
    h = jnp.maximum(h, 0.0)
    h = jnp.maximum(jnp.dot(h, w1_ref[...], preferred_element_type=_F32)
                    + b1_ref[...], 0.0)
    h = jnp.dot(h, w2_ref[...], preferred_element_type=_F32) + b2_ref[...]
    xn = _ln_res(x, h, g_ref[...], be_ref[...])
    xn_ref[...] = xn
    if has_next:
        p_ref[...] = jnp.dot(xn, ws_ref[...], preferred_element_type=_F32)
        q_ref[...] = jnp.dot(xn, wd_ref[...], preferred_element_type=_F32)


def _node_call(x, aggs, w0x, w0a, b0, w1, b1, w2, b2, g, be, ws_next,
               wd_next):
    grid = (N // TN,)
    row = pl.BlockSpec((TN, D), lambda i: (i, 0))
    wfull = pl.BlockSpec((H, H), lambda i: (0, 0))
    vfull = pl.BlockSpec((1, H), lambda i: (0, 0))
    has_next = ws_next is not None
    ins = [x] + aggs + [w0x, w0a, b0, w1, b1, w2, b2, g, be]
    in_specs = [row, row, row, row, row, wfull, wfull, vfull, wfull, vfull,
                wfull, vfull, vfull, vfull]
    if has_next:
        ins += [ws_next, wd_next]
        in_specs += [wfull, wfull]
        out_specs = [row, row, row]
        out_shape = [jax.ShapeDtypeStruct((N, D), _F32),
                     jax.ShapeDtypeStruct((N, H), _F32),
                     jax.ShapeDtypeStruct((N, H), _F32)]
    else:
        out_specs = row
        out_shape = jax.ShapeDtypeStruct((N, D), _F32)
    return pl.pallas_call(
        functools.partial(_node_body, has_next),
        grid=grid,
        in_specs=in_specs,
        out_specs=out_specs,
        out_shape=out_shape,
    )(*ins)


def kernel(x, edge_index, edge_attr, We0, be0, We1, be1, We2, be2, eg, ebe,
           Wn0, bn0, Wn1, bn1, Wn2, bn2, ng, nbe):
    nchunk, gw, gew, _ = _group_geom(ESEG)
    gpad = gew * NW_SC
    src_segs, dst_segs, dstr_segs, e_segs = [], [], [], []
    for s in range(NSEG):
        src_s = edge_index[0, s * ESEG:(s + 1) * ESEG]
        dst_s = edge_index[1, s * ESEG:(s + 1) * ESEG]
        src_segs.append(jnp.pad(src_s, (0, gpad - ESEG)))
        dst_p = jnp.pad(dst_s, (0, gpad - ESEG))
        dst_segs.append(dst_p)
        dstr_segs.append(dst_p.reshape(NW_SC, gw, CH))
        e_segs.append(edge_attr[s * ESEG:(s + 1) * ESEG])

    p, q = _pq_call(x, We0[0, :D], We0[0, D:2 * D])
    for i in range(NB):
        pres = [_sc_gather_seg(p, q, src_segs[s], dst_segs[s])
                for s in range(NSEG)]
        e_segs = [_edge_call(pres[s], e_segs[s], We0[i, 2 * D:],
                             be0[i][None], We1[i], be1[i][None], We2[i],
                             be2[i][None], eg[i][None], ebe[i][None])
                  for s in range(NSEG)]
        parts = [_sc_scatter_seg(e_segs[s], dstr_segs[s])
                 for s in range(NSEG)]
        aggs = [parts[0][0], parts[0][1], parts[1][0], parts[1][1]]
        ws_next = We0[i + 1, :D] if i + 1 < NB else None
        wd_next = We0[i + 1, D:2 * D] if i + 1 < NB else None
        out = _node_call(x, aggs, Wn0[i, :D], Wn0[i, D:], bn0[i][None],
                         Wn1[i], bn1[i][None], Wn2[i], bn2[i][None],
                         ng[i][None], nbe[i][None], ws_next, wd_next)
        if i + 1 < NB:
            x, p, q = out
        else:
            x = out
    return x


# speedup vs baseline: 4.7486x; 1.0020x over previous
import functools

import jax
import jax.numpy as jnp
from jax import lax
from jax.experimental import pallas as pl
from jax.experimental.pallas import tpu as pltpu
from jax.experimental.pallas import tpu_sc as plsc

N = 10000
E = 160000
D = 128
H = 128
NB = 3

TN = 2000
TE = 2000
_F32 = jnp.float32

NC_SC = 2
NS_SC = 16
NW_SC = NC_SC * NS_SC
CH = 128
N_PAD = 10240
NROW_S = N_PAD // NS_SC
RB = 128

NSEG = 2
ESEG = E // NSEG

_sc_mesh = plsc.VectorSubcoreMesh(core_axis_name="c", subcore_axis_name="s",
                                  num_cores=NC_SC, num_subcores=NS_SC)


def _group_geom(esz):
    nchunk = esz // CH
    gw = -(-nchunk // NW_SC)
    t_last = nchunk - gw * (NW_SC - 1)
    return nchunk, gw, gw * CH, t_last


def _mk_gather(esz):
    nchunk, gw, gew, t_last = _group_geom(esz)

    @functools.partial(
        pl.kernel,
        out_type=jax.ShapeDtypeStruct((esz, D), _F32),
        mesh=_sc_mesh,
        scratch_types=[
            pltpu.VMEM((gew,), jnp.int32),
            pltpu.VMEM((gew,), jnp.int32),
            pltpu.VMEM((2, CH, D), _F32),
            pltpu.VMEM((2, CH, D), _F32),
            pltpu.VMEM((2, CH, D), _F32),
            pltpu.SemaphoreType.DMA,
            pltpu.SemaphoreType.DMA,
            pltpu.SemaphoreType.DMA,
            pltpu.SemaphoreType.DMA,
            pltpu.SemaphoreType.DMA,
        ],
    )
    def gather(p_hbm, q_hbm, src_hbm, dst_hbm, out_hbm, si, di, rs, rd, wb,
               gs0, gs1, ws0, ws1, isem):
        wid = lax.axis_index("s") * NC_SC + lax.axis_index("c")
        t_w = jnp.where(wid == NW_SC - 1, t_last, gw)
        ebase = wid * gew
        c1 = pltpu.async_copy(src_hbm.at[pl.ds(ebase, gew)], si, isem)
        c2 = pltpu.async_copy(dst_hbm.at[pl.ds(ebase, gew)], di, isem)
        c1.wait()
        c2.wait()

        gsems = (gs0, gs1)
        wsems = (ws0, ws1)

        def fire(c, b):
            off = pl.multiple_of(c * CH, CH)
            pltpu.async_copy(p_hbm.at[si.at[pl.ds(off, CH)]], rs.at[b],
                             gsems[b])
            pltpu.async_copy(q_hbm.at[di.at[pl.ds(off, CH)]], rd.at[b],
                             gsems[b])

        def wait_g(b):
            pltpu.make_async_copy(p_hbm.at[pl.ds(0, CH)], rs.at[b],
                                  gsems[b]).wait()
            pltpu.make_async_copy(q_hbm.at[pl.ds(0, CH)], rd.at[b],
                                  gsems[b]).wait()

        def add(b):
            rb = rs.at[b]
            db = rd.at[b]
            ob = wb.at[b]

            def row(r, carry2):
                for j in range(D // 16):
                    sl = pl.ds(j * 16, 16)
                    ob[r, sl] = rb[r, sl] + db[r, sl]
                return carry2

            lax.fori_loop(0, CH, row, 0)

        def fire_write(c, b):
            off = pl.multiple_of(c * CH, CH)
            pltpu.async_copy(wb.at[b], out_hbm.at[pl.ds(ebase + off, CH)],
                             wsems[b])

        def wait_w(b):
            pltpu.make_async_copy(wb.at[b], out_hbm.at[pl.ds(0, CH)],
                                  wsems[b]).wait()

        fire(0, 0)
        fire(1, 1)

        def pair(k2, carry):
            for b in (0, 1):
                c = 2 * k2 + b
                wait_g(b)

                @pl.when(c >= 2)
                def _():
                    wait_w(b)

                add(b)
                nxt = c + 2

                @pl.when(nxt < t_w)
                def _():
                    fire(nxt, b)

                fire_write(c, b)
            return carry

        lax.fori_loop(0, t_w // 2, pair, 0)

        @pl.when(t_w % 2 == 1)
        def _():
            wait_g(0)
            wait_w(0)
            add(0)
            fire_write(t_w - 1, 0)

        wait_w(0)
        wait_w(1)

    return gather


def _mk_scatter(esz):
    nchunk, gw, gew, t_last = _group_geom(esz)

    @functools.partial(
        pl.kernel,
        out_type=jax.ShapeDtypeStruct((NC_SC, N_PAD, D), _F32),
        mesh=_sc_mesh,
        scratch_types=[
            pltpu.VMEM((gw, CH), jnp.int32),
            pltpu.VMEM((2, CH, D), _F32),
            pltpu.VMEM_SHARED((N_PAD, D), _F32),
            pltpu.SemaphoreType.DMA,
            pltpu.SemaphoreType.DMA,
            pltpu.SemaphoreType.DMA,
            pltpu.SemaphoreType.DMA,
            pltpu.SemaphoreType.DMA,
        ],
    )
    def scatter(e_hbm, dstr_hbm, out_hbm, di, rows, acc_sh,
                fs0, fs1, ss0, ss1, osem):
        cid = lax.axis_index("c")
        sid = lax.axis_index("s")
        wid = sid * NC_SC + cid
        t_w = jnp.where(wid == NW_SC - 1, t_last, gw)
        cbase = wid * gw

        ic = pltpu.async_copy(dstr_hbm.at[wid], di, osem)

        zb = rows.at[0]

        def zrow(r, carry):
            for j in range(D // 16):
                zb[r, pl.ds(j * 16, 16)] = jnp.zeros((16,), _F32)
            return carry

        lax.fori_loop(0, RB, zrow, 0)
        for t in range(NROW_S // RB):
            pltpu.sync_copy(zb, acc_sh.at[pl.ds(sid * NROW_S + t * RB, RB)])
        ic.wait()
        plsc.subcore_barrier()

        fsems = (fs0, fs1)
        ssems = (ss0, ss1)

        def fire_fill(c, b):
            base = pl.multiple_of((cbase + c) * CH, CH)
            pltpu.async_copy(e_hbm.at[pl.ds(base, CH)], rows.at[b], fsems[b])

        def wait_fill(b):
            pltpu.make_async_copy(e_hbm.at[pl.ds(0, CH)], rows.at[b],
                                  fsems[b]).wait()

        def wait_scat(b):
            pltpu.make_async_copy(rows.at[b], acc_sh.at[pl.ds(0, CH)],
                                  ssems[b]).wait()

        fire_fill(0, 0)

        def pair(k2, carry):
            for b in (0, 1):
                c = 2 * k2 + b
                wait_fill(b)
                pltpu.async_copy(rows.at[b], acc_sh.at[di.at[c]], ssems[b],
                                 add=True)
                nb = 1 - b

                @pl.when(c + 1 < t_w)
                def _():
                    @pl.when(c >= 1)
                    def _():
                        wait_scat(nb)

                    fire_fill(c + 1, nb)
            return carry

        lax.fori_loop(0, t_w // 2, pair, 0)

        @pl.when(t_w % 2 == 1)
        def _():
            wait_fill(0)
            pltpu.async_copy(rows.at[0], acc_sh.at[di.at[t_w - 1]], ss0,
                             add=True)

        wait_scat(0)
        wait_scat(1)
        plsc.subcore_barrier()

        nhop = NROW_S // RB
        for t in range(nhop):
            b = t % 2
            if t >= 2:
                pltpu.make_async_copy(rows.at[b],
                                      out_hbm.at[0, pl.ds(0, RB)],
                                      osem).wait()
            row = sid * NROW_S + t * RB
            pltpu.sync_copy(acc_sh.at[pl.ds(row, RB)], rows.at[b])
            pltpu.async_copy(rows.at[b], out_hbm.at[cid, pl.ds(row, RB)],
                             osem)
        for t in range(min(2, nhop)):
            pltpu.make_async_copy(rows.at[t], out_hbm.at[0, pl.ds(0, RB)],
                                  osem).wait()

    return scatter


_sc_gather_seg = _mk_gather(ESEG)
_sc_scatter_seg = _mk_scatter(ESEG)


def _ln_res(base, h, g, b):
    mu = jnp.mean(h, axis=-1, keepdims=True)
    v = jnp.mean((h - mu) ** 2, axis=-1, keepdims=True)
    return base + (h - mu) / jnp.sqrt(v + 1e-5) * g + b


def _pq_body(x_ref, ws_ref, wd_ref, p_ref, q_ref):
    x = x_ref[...]
    p_ref[...] = jnp.dot(x, ws_ref[...], preferred_element_type=_F32)
    q_ref[...] = jnp.dot(x, wd_ref[...], preferred_element_type=_F32)


def _pq_call(x, ws, wd):
    grid = (N // TN,)
    row = pl.BlockSpec((TN, D), lambda i: (i, 0))
    full = pl.BlockSpec((D, H), lambda i: (0, 0))
    return pl.pallas_call(
        _pq_body,
        grid=grid,
        in_specs=[row, full, full],
        out_specs=[row, row],
        out_shape=[jax.ShapeDtypeStruct((N, H), _F32),
                   jax.ShapeDtypeStruct((N, H), _F32)],
    )(x, ws, wd)


def _edge_body(pre_ref, e_ref, w0_ref, b0_ref, w1_ref, b1_ref, w2_ref,
               b2_ref, g_ref, be_ref, out_ref):
    e = e_ref[...]
    h = jnp.dot(e, w0_ref[...], preferred_element_type=_F32)
    h = jnp.maximum(h + pre_ref[...] + b0_ref[...], 0.0)
    h = jnp.maximum(jnp.dot(h, w1_ref[...], preferred_element_type=_F32)
                    + b1_ref[...], 0.0)
    h = jnp.dot(h, w2_ref[...], preferred_element_type=_F32) + b2_ref[...]
    out_ref[...] = _ln_res(e, h, g_ref[...], be_ref[...])


def _edge_call(pre, e, w0, b0, w1, b1, w2, b2, g, be):
    esz = e.shape[0]
    grid = (esz // TE,)
    row = pl.BlockSpec((TE, D), lambda i: (i, 0))
    wfull = pl.BlockSpec((H, H), lambda i: (0, 0))
    vfull = pl.BlockSpec((1, H), lambda i: (0, 0))
    return pl.pallas_call(
        _edge_body,
        grid=grid,
        in_specs=[row, row, wfull, vfull, wfull, vfull, wfull, vfull,
                  vfull, vfull],
        out_specs=row,
        out_shape=jax.ShapeDtypeStruct((esz, D), _F32),
    )(pre, e, w0, b0, w1, b1, w2, b2, g, be)


def _node_body(has_next, x_ref, a0_ref, a1_ref, a2_ref, a3_ref, w0x_ref,
               w0a_ref, b0_ref, w1_ref, b1_ref, w2_ref, b2_ref, g_ref,
               be_ref, *rest):
    if has_next:
        ws_ref, wd_ref, xn_ref, p_ref, q_ref = rest
    else:
        (xn_ref,) = rest
    x = x_ref[...]
    agg = ((a0_ref[...] + a1_ref[...]) + (a2_ref[...] + a3_ref[...]))
    h = (jnp.dot(x, w0x_ref[...], preferred_element_type=_F32)
         + jnp.dot(agg, w0a_ref[...], preferred_element_type=_F32)
         + b0_ref[...])
    h = jnp.maximum(h, 0.0)
    h = jnp.maximum(jnp.dot(h, w1_ref[...], preferred_element_type=_F32)
                    + b1_ref[...], 0.0)
    h = jnp.dot(h, w2_ref[...], preferred_element_type=_F32) + b2_ref[...]
    xn = _ln_res(x, h, g_ref[...], be_ref[...])
    xn_ref[...] = xn
    if has_next:
        p_ref[...] = jnp.dot(xn, ws_ref[...], preferred_element_type=_F32)
        q_ref[...] = jnp.dot(xn, wd_ref[...], preferred_element_type=_F32)


def _node_call(x, aggs, w0x, w0a, b0, w1, b1, w2, b2, g, be, ws_next,
               wd_next):
    grid = (N // TN,)
    row = pl.BlockSpec((TN, D), lambda i: (i, 0))
    wfull = pl.BlockSpec((H, H), lambda i: (0, 0))
    vfull = pl.BlockSpec((1, H), lambda i: (0, 0))
    has_next = ws_next is not None
    ins = [x] + aggs + [w0x, w0a, b0, w1, b1, w2, b2, g, be]
    in_specs = [row, row, row, row, row, wfull, wfull, vfull, wfull, vfull,
                wfull, vfull, vfull, vfull]
    if has_next:
        ins += [ws_next, wd_next]
        in_specs += [wfull, wfull]
        out_specs = [row, row, row]
        out_shape = [jax.ShapeDtypeStruct((N, D), _F32),
                     jax.ShapeDtypeStruct((N, H), _F32),
                     jax.ShapeDtypeStruct((N, H), _F32)]
    else:
        out_specs = row
        out_shape = jax.ShapeDtypeStruct((N, D), _F32)
    return pl.pallas_call(
        functools.partial(_node_body, has_next),
        grid=grid,
        in_specs=in_specs,
        out_specs=out_specs,
        out_shape=out_shape,
    )(*ins)


def kernel(x, edge_index, edge_attr, We0, be0, We1, be1, We2, be2, eg, ebe,
           Wn0, bn0, Wn1, bn1, Wn2, bn2, ng, nbe):
    nchunk, gw, gew, _ = _group_geom(ESEG)
    gpad = gew * NW_SC
    src_segs, dst_segs, dstr_segs, e_segs = [], [], [], []
    for s in range(NSEG):
        src_s = edge_index[0, s * ESEG:(s + 1) * ESEG]
        dst_s = edge_index[1, s * ESEG:(s + 1) * ESEG]
        src_segs.append(jnp.pad(src_s, (0, gpad - ESEG)))
        dst_p = jnp.pad(dst_s, (0, gpad - ESEG))
        dst_segs.append(dst_p)
        dstr_segs.append(dst_p.reshape(NW_SC, gw, CH))
        e_segs.append(edge_attr[s * ESEG:(s + 1) * ESEG])

    p, q = _pq_call(x, We0[0, :D], We0[0, D:2 * D])
    for i in range(NB):
        pres = [_sc_gather_seg(p, q, src_segs[s], dst_segs[s])
                for s in range(NSEG)]
        e_segs = [_edge_call(pres[s], e_segs[s], We0[i, 2 * D:],
                             be0[i][None], We1[i], be1[i][None], We2[i],
                             be2[i][None], eg[i][None], ebe[i][None])
                  for s in range(NSEG)]
        parts = [_sc_scatter_seg(e_segs[s], dstr_segs[s])
                 for s in range(NSEG)]
        aggs = [parts[0][0], parts[0][1], parts[1][0], parts[1][1]]
        ws_next = We0[i + 1, :D] if i + 1 < NB else None
        wd_next = We0[i + 1, D:2 * D] if i + 1 < NB else None
        out = _node_call(x, aggs, Wn0[i, :D], Wn0[i, D:], bn0[i][None],
                         Wn1[i], bn1[i][None], Wn2[i], bn2[i][None],
                         ng[i][None], nbe[i][None], ws_next, wd_next)
        if i + 1 < NB:
            x, p, q = out
        else:
            x = out
    return x
